# baseline (speedup 1.0000x reference)
import functools

import jax
import jax.numpy as jnp
from jax import lax
from jax.experimental import pallas as pl
from jax.experimental.pallas import tpu as pltpu
from jax.experimental.pallas import tpu_sc as plsc

EMB = 64
IDX_ROW = 128
ROWS_PER_CHUNK = 4
CHUNK = IDX_ROW * ROWS_PER_CHUNK


@functools.cache
def _build(n_idx: int):
    info = plsc.get_sparse_core_info()
    nw = info.num_cores * info.num_subcores
    n_chunks = n_idx // CHUNK
    assert n_chunks * CHUNK == n_idx

    mesh = plsc.VectorSubcoreMesh(core_axis_name="c", subcore_axis_name="s")
    out_t = jax.ShapeDtypeStruct((n_idx, EMB), jnp.float32)

    @functools.partial(
        pl.kernel,
        mesh=mesh,
        out_type=[out_t, out_t],
        scratch_types=[
            pltpu.VMEM((2, ROWS_PER_CHUNK, IDX_ROW), jnp.int32),
            pltpu.VMEM((2, CHUNK, EMB), jnp.float32),
            pltpu.SemaphoreType.DMA,
            pltpu.SemaphoreType.DMA,
            pltpu.SemaphoreType.DMA,
            pltpu.SemaphoreType.DMA,
            pltpu.SemaphoreType.DMA,
            pltpu.SemaphoreType.DMA,
        ],
        compiler_params=pltpu.CompilerParams(use_tc_tiling_on_sc=False),
    )
    def k(w_seq, w_exp, seq_idx, exp_idx, o_seq, o_exp, idx_v, rows_v,
          si0, si1, sg0, sg1, so0, so1):
        wid = lax.axis_index("s") * info.num_cores + lax.axis_index("c")
        sem_i, sem_g, sem_o = [si0, si1], [sg0, sg1], [so0, so1]

        def start_idx(idx_hbm, b, c):
            pltpu.async_copy(
                idx_hbm.at[pl.ds(c * ROWS_PER_CHUNK, ROWS_PER_CHUNK)],
                idx_v.at[b], sem_i[b])

        def gather_cps(w_hbm, b):
            return [
                pltpu.make_async_copy(
                    w_hbm.at[idx_v.at[b].at[i]],
                    rows_v.at[b].at[pl.ds(i * IDX_ROW, IDX_ROW)],
                    sem_g[b])
                for i in range(ROWS_PER_CHUNK)
            ]

        def wait_store(out_hbm, b):
            pltpu.make_async_copy(
                rows_v.at[b], out_hbm.at[pl.ds(0, CHUNK)], sem_o[b]).wait()

        def do_table(idx_hbm, w_hbm, out_hbm, first):
            n_mine = (n_chunks - wid + nw - 1) // nw

            for b in range(2):
                @pl.when(b < n_mine)
                def _():
                    start_idx(idx_hbm, b, wid + b * nw)

            def pair_body(p, carry):
                for b in range(2):
                    j = 2 * p + b

                    @pl.when(j < n_mine)
                    def _():
                        pltpu.make_async_copy(
                            idx_hbm.at[pl.ds(0, ROWS_PER_CHUNK)],
                            idx_v.at[b], sem_i[b]).wait()
                        @pl.when(j >= 2)
                        def _():
                            wait_store(out_hbm, b)
                        for cp in gather_cps(w_hbm, b):
                            cp.start()

                for b in range(2):
                    j = 2 * p + b

                    @pl.when(j < n_mine)
                    def _():
                        for cp in gather_cps(w_hbm, b):
                            cp.wait()
                        c = wid + j * nw
                        pltpu.async_copy(
                            rows_v.at[b], out_hbm.at[pl.ds(c * CHUNK, CHUNK)],
                            sem_o[b])

                        @pl.when(j + 2 < n_mine)
                        def _():
                            start_idx(idx_hbm, b, wid + (j + 2) * nw)
                return carry

            lax.fori_loop(0, (n_mine + 1) // 2, pair_body, 0)

            for b in range(2):
                @pl.when(b < n_mine)
                def _():
                    wait_store(out_hbm, b)

        do_table(seq_idx, w_seq, o_seq, True)
        do_table(exp_idx, w_exp, o_exp, False)

    return k


def kernel(seqs, exps, W_seq, W_exp):
    b, s = seqs.shape
    n_idx = b * s
    seq_idx = seqs.reshape(n_idx // IDX_ROW, IDX_ROW).astype(jnp.int32)
    exp_idx = exps.reshape(n_idx // IDX_ROW, IDX_ROW).astype(jnp.int32)
    o_seq, o_exp = _build(n_idx)(W_seq, W_exp, seq_idx, exp_idx)
    return (o_seq.reshape(b, s, EMB), o_exp.reshape(b, s, EMB))


# speedup vs baseline: 1.4981x; 1.4981x over previous
import functools

import jax
import jax.numpy as jnp
from jax import lax
from jax.experimental import pallas as pl
from jax.experimental.pallas import tpu as pltpu
from jax.experimental.pallas import tpu_sc as plsc

VOCAB = 457
EMB = 64
IDX_ROW = 128
ROWS_PER_CHUNK = 4
CHUNK = IDX_ROW * ROWS_PER_CHUNK


@functools.cache
def _build(n_idx: int):
    info = plsc.get_sparse_core_info()
    nw = info.num_cores * info.num_subcores
    n_chunks = n_idx // CHUNK
    assert n_chunks * CHUNK == n_idx

    mesh = plsc.VectorSubcoreMesh(core_axis_name="c", subcore_axis_name="s")
    out_t = jax.ShapeDtypeStruct((n_idx, EMB), jnp.float32)

    @functools.partial(
        pl.kernel,
        mesh=mesh,
        out_type=[out_t, out_t],
        scratch_types=[
            pltpu.VMEM((2, ROWS_PER_CHUNK, IDX_ROW), jnp.int32),
            pltpu.VMEM((2, CHUNK, EMB), jnp.float32),
            pltpu.VMEM_SHARED((VOCAB, EMB), jnp.float32),
            pltpu.VMEM_SHARED((VOCAB, EMB), jnp.float32),
            pltpu.SemaphoreType.DMA,
            pltpu.SemaphoreType.DMA,
            pltpu.SemaphoreType.DMA,
            pltpu.SemaphoreType.DMA,
            pltpu.SemaphoreType.DMA,
            pltpu.SemaphoreType.DMA,
        ],
        compiler_params=pltpu.CompilerParams(use_tc_tiling_on_sc=False),
    )
    def k(w_seq, w_exp, seq_idx, exp_idx, o_seq, o_exp, idx_v, rows_v,
          w_seq_v, w_exp_v, si0, si1, sg0, sg1, so0, so1):
        wid = lax.axis_index("s") * info.num_cores + lax.axis_index("c")
        sem_i, sem_g, sem_o = [si0, si1], [sg0, sg1], [so0, so1]
        @pl.when(lax.axis_index("s") == 0)
        def _():
            pltpu.sync_copy(w_seq, w_seq_v)
            pltpu.sync_copy(w_exp, w_exp_v)
        plsc.subcore_barrier()

        def start_idx(idx_hbm, b, c):
            pltpu.async_copy(
                idx_hbm.at[pl.ds(c * ROWS_PER_CHUNK, ROWS_PER_CHUNK)],
                idx_v.at[b], sem_i[b])

        def gather_cps(w_hbm, b):
            return [
                pltpu.make_async_copy(
                    w_hbm.at[idx_v.at[b].at[i]],
                    rows_v.at[b].at[pl.ds(i * IDX_ROW, IDX_ROW)],
                    sem_g[b])
                for i in range(ROWS_PER_CHUNK)
            ]

        def wait_store(out_hbm, b):
            pltpu.make_async_copy(
                rows_v.at[b], out_hbm.at[pl.ds(0, CHUNK)], sem_o[b]).wait()

        def do_table(idx_hbm, w_hbm, out_hbm, first):
            n_mine = (n_chunks - wid + nw - 1) // nw

            for b in range(2):
                @pl.when(b < n_mine)
                def _():
                    start_idx(idx_hbm, b, wid + b * nw)

            def pair_body(p, carry):
                for b in range(2):
                    j = 2 * p + b

                    @pl.when(j < n_mine)
                    def _():
                        pltpu.make_async_copy(
                            idx_hbm.at[pl.ds(0, ROWS_PER_CHUNK)],
                            idx_v.at[b], sem_i[b]).wait()
                        @pl.when(j >= 2)
                        def _():
                            wait_store(out_hbm, b)
                        for cp in gather_cps(w_hbm, b):
                            cp.start()

                for b in range(2):
                    j = 2 * p + b

                    @pl.when(j < n_mine)
                    def _():
                        for cp in gather_cps(w_hbm, b):
                            cp.wait()
                        c = wid + j * nw
                        pltpu.async_copy(
                            rows_v.at[b], out_hbm.at[pl.ds(c * CHUNK, CHUNK)],
                            sem_o[b])

                        @pl.when(j + 2 < n_mine)
                        def _():
                            start_idx(idx_hbm, b, wid + (j + 2) * nw)
                return carry

            lax.fori_loop(0, (n_mine + 1) // 2, pair_body, 0)

            for b in range(2):
                @pl.when(b < n_mine)
                def _():
                    wait_store(out_hbm, b)

        do_table(seq_idx, w_seq_v, o_seq, True)
        do_table(exp_idx, w_exp_v, o_exp, False)

    return k


def kernel(seqs, exps, W_seq, W_exp):
    b, s = seqs.shape
    n_idx = b * s
    seq_idx = seqs.reshape(n_idx // IDX_ROW, IDX_ROW).astype(jnp.int32)
    exp_idx = exps.reshape(n_idx // IDX_ROW, IDX_ROW).astype(jnp.int32)
    o_seq, o_exp = _build(n_idx)(W_seq, W_exp, seq_idx, exp_idx)
    return (o_seq.reshape(b, s, EMB), o_exp.reshape(b, s, EMB))


# speedup vs baseline: 1.6298x; 1.0879x over previous
import functools

import jax
import jax.numpy as jnp
from jax import lax
from jax.experimental import pallas as pl
from jax.experimental.pallas import tpu as pltpu
from jax.experimental.pallas import tpu_sc as plsc

VOCAB = 457
EMB = 64
GATHER_UNIT = 128


@functools.cache
def _build(batch: int, seq: int):
    info = plsc.get_sparse_core_info()
    nw = info.num_cores * info.num_subcores
    rows_per_w = batch // nw
    assert rows_per_w * nw == batch
    splits = []
    off = 0
    while off < seq:
        n = min(GATHER_UNIT, seq - off)
        splits.append((off, n))
        off += n

    mesh = plsc.VectorSubcoreMesh(core_axis_name="c", subcore_axis_name="s")
    out_t = jax.ShapeDtypeStruct((batch, seq, EMB), jnp.float32)

    @functools.partial(
        pl.kernel,
        mesh=mesh,
        out_type=[out_t, out_t],
        scratch_types=[
            pltpu.VMEM((2, seq), jnp.int32),
            pltpu.VMEM((2, seq, EMB), jnp.float32),
            pltpu.VMEM_SHARED((VOCAB, EMB), jnp.float32),
            pltpu.VMEM_SHARED((VOCAB, EMB), jnp.float32),
            pltpu.SemaphoreType.DMA,
            pltpu.SemaphoreType.DMA,
            pltpu.SemaphoreType.DMA,
            pltpu.SemaphoreType.DMA,
            pltpu.SemaphoreType.DMA,
            pltpu.SemaphoreType.DMA,
        ],
    )
    def k(w_seq, w_exp, seq_idx, exp_idx, o_seq, o_exp, idx_v, rows_v,
          w_seq_s, w_exp_s, si0, si1, sg0, sg1, so0, so1):
        wid = lax.axis_index("s") * info.num_cores + lax.axis_index("c")
        sem_i, sem_g, sem_o = [si0, si1], [sg0, sg1], [so0, so1]

        @pl.when(lax.axis_index("s") == 0)
        def _():
            pltpu.sync_copy(w_seq, w_seq_s)
            pltpu.sync_copy(w_exp, w_exp_s)
        plsc.subcore_barrier()

        def start_idx(idx_hbm, b, row):
            pltpu.async_copy(idx_hbm.at[row], idx_v.at[b], sem_i[b])

        def gather_cps(w_s, b):
            return [
                pltpu.make_async_copy(
                    w_s.at[idx_v.at[b].at[pl.ds(off, n)]],
                    rows_v.at[b].at[pl.ds(off, n)],
                    sem_g[b])
                for off, n in splits
            ]

        def wait_store(out_hbm, b):
            pltpu.make_async_copy(rows_v.at[b], out_hbm.at[0], sem_o[b]).wait()

        def do_table(idx_hbm, w_s, out_hbm):
            base = wid * rows_per_w

            for b in range(2):
                start_idx(idx_hbm, b, base + b)

            def pair_body(p, carry):
                for b in range(2):
                    j = 2 * p + b
                    pltpu.make_async_copy(
                        idx_hbm.at[0], idx_v.at[b], sem_i[b]).wait()

                    @pl.when(j >= 2)
                    def _():
                        wait_store(out_hbm, b)
                    for cp in gather_cps(w_s, b):
                        cp.start()

                for b in range(2):
                    j = 2 * p + b
                    for cp in gather_cps(w_s, b):
                        cp.wait()
                    pltpu.async_copy(rows_v.at[b], out_hbm.at[base + j], sem_o[b])

                    @pl.when(j + 2 < rows_per_w)
                    def _():
                        start_idx(idx_hbm, b, base + j + 2)
                return carry

            lax.fori_loop(0, rows_per_w // 2, pair_body, 0)

            for b in range(2):
                wait_store(out_hbm, b)

        do_table(seq_idx, w_seq_s, o_seq)
        do_table(exp_idx, w_exp_s, o_exp)

    return k


def kernel(seqs, exps, W_seq, W_exp):
    b, s = seqs.shape
    o_seq, o_exp = _build(b, s)(
        W_seq, W_exp, seqs.astype(jnp.int32), exps.astype(jnp.int32))
    return (o_seq, o_exp)
